# baseline (speedup 1.0000x reference)
import functools

import jax
import jax.numpy as jnp
from jax import lax
from jax.experimental import pallas as pl
from jax.experimental.pallas import tpu as pltpu
from jax.experimental.pallas import tpu_sc as plsc

B = 16384
K = 64
NOUT = 256
NC = 2
NS = 16
NW = NC * NS
BPW = B // NW
LANES = 16


def _sc_gather_prod_kernel(idx0_hbm, idx1_hbm, idx2_hbm, t0_hbm, t1_hbm,
                           t2_hbm, out_hbm, i0_v, i1_v, i2_v, r0_v, r1_v,
                           r2_v, s0, s1, s2):
    wid = lax.axis_index("s") * NC + lax.axis_index("c")
    base = wid * BPW
    pltpu.sync_copy(idx0_hbm.at[pl.ds(base, BPW)], i0_v)
    pltpu.sync_copy(idx1_hbm.at[pl.ds(base, BPW)], i1_v)
    pltpu.sync_copy(idx2_hbm.at[pl.ds(base, BPW)], i2_v)
    c0 = pltpu.async_copy(t0_hbm.at[i0_v], r0_v, s0)
    c1 = pltpu.async_copy(t1_hbm.at[i1_v], r1_v, s1)
    c2 = pltpu.async_copy(t2_hbm.at[i2_v], r2_v, s2)
    c0.wait()
    c1.wait()
    c2.wait()

    def body(r, carry):
        for c in range(K // LANES):
            sl = pl.ds(c * LANES, LANES)
            r0_v[r, sl] = r0_v[r, sl] * r1_v[r, sl] * r2_v[r, sl]
        return carry

    lax.fori_loop(0, BPW, body, 0, unroll=4)
    pltpu.sync_copy(r0_v, out_hbm.at[pl.ds(base, BPW)])


@jax.jit
def _sc_gather_prod(idx0, idx1, idx2, f0, f1, f2):
    mesh = plsc.VectorSubcoreMesh(core_axis_name="c", subcore_axis_name="s")
    return pl.kernel(
        _sc_gather_prod_kernel,
        mesh=mesh,
        compiler_params=pltpu.CompilerParams(use_tc_tiling_on_sc=False),
        out_type=jax.ShapeDtypeStruct((B, K), jnp.float32),
        scratch_types=[
            pltpu.VMEM((BPW,), jnp.int32),
            pltpu.VMEM((BPW,), jnp.int32),
            pltpu.VMEM((BPW,), jnp.int32),
            pltpu.VMEM((BPW, K), jnp.float32),
            pltpu.VMEM((BPW, K), jnp.float32),
            pltpu.VMEM((BPW, K), jnp.float32),
            pltpu.SemaphoreType.DMA,
            pltpu.SemaphoreType.DMA,
            pltpu.SemaphoreType.DMA,
        ],
    )(idx0, idx1, idx2, f0, f1, f2)


BM = 1024


def _tc_proj_kernel(prod_ref, f3_ref, ls_ref, out_ref, ls_out_ref):
    out_ref[...] = lax.dot_general(
        prod_ref[...], f3_ref[...],
        dimension_numbers=(((1,), (1,)), ((), ())),
        preferred_element_type=jnp.float32,
    )
    ls_out_ref[...] = jnp.clip(ls_ref[...], -2.5, 0.0)


@jax.jit
def _tc_proj(prod, f3, log_sigma):
    return pl.pallas_call(
        _tc_proj_kernel,
        grid=(B // BM,),
        in_specs=[
            pl.BlockSpec((BM, K), lambda i: (i, 0)),
            pl.BlockSpec((NOUT, K), lambda i: (0, 0)),
            pl.BlockSpec((1, NOUT), lambda i: (0, 0)),
        ],
        out_specs=[
            pl.BlockSpec((BM, NOUT), lambda i: (i, 0)),
            pl.BlockSpec((1, NOUT), lambda i: (0, 0)),
        ],
        out_shape=[
            jax.ShapeDtypeStruct((B, NOUT), jnp.float32),
            jax.ShapeDtypeStruct((1, NOUT), jnp.float32),
        ],
    )(prod, f3, log_sigma)


def kernel(indices, f0, f1, f2, f3, log_sigma):
    idx = indices.astype(jnp.int32)
    idx0 = idx[:, 0]
    idx1 = idx[:, 1]
    idx2 = idx[:, 2]
    prod = _sc_gather_prod(idx0, idx1, idx2, f0, f1, f2)
    res, ls = _tc_proj(prod, f3, log_sigma)
    return (res, ls)
